# baseline (speedup 1.0000x reference)
import functools

import jax
import jax.numpy as jnp
from jax import lax
from jax.experimental import pallas as pl
from jax.experimental.pallas import tpu as pltpu
from jax.experimental.pallas import tpu_sc as plsc

NUM_ROWS = 1_000_000
DIM = 64
NA, NT = 4096, 200
BATCH = NA * NT
NC, NS = 2, 16
NW = NC * NS
CHUNK = 128
NCH = BATCH // NW // CHUNK
OPC = CHUNK // 2
PFR = CHUNK // 8
NBUF = 2

_mesh = plsc.VectorSubcoreMesh(core_axis_name="c", subcore_axis_name="s")


@functools.partial(
    pl.kernel,
    mesh=_mesh,
    out_type=jax.ShapeDtypeStruct((BATCH // 2, 2 * DIM), jnp.float32),
    scratch_types=[
        pltpu.VMEM((NBUF, CHUNK), jnp.int32),
        pltpu.VMEM((PFR, CHUNK), jnp.float32),
        pltpu.VMEM((PFR, CHUNK), jnp.float32),
        pltpu.VMEM((CHUNK, 2 * DIM), jnp.float32),
        pltpu.VMEM((CHUNK, 2 * DIM), jnp.float32),
        pltpu.VMEM((OPC, 2 * DIM), jnp.float32),
        pltpu.VMEM((OPC, 2 * DIM), jnp.float32),
        pltpu.SemaphoreType.DMA,
        pltpu.SemaphoreType.DMA,
        pltpu.SemaphoreType.DMA,
        pltpu.SemaphoreType.DMA,
        pltpu.SemaphoreType.DMA,
        pltpu.SemaphoreType.DMA,
    ],
)
def _gather(
    xs_hbm, pf_hbm, wp_hbm, out_hbm,
    jbuf, pf0, pf1, r0, r1, ob0, ob1,
    sj0, sj1, sg0, sg1, so0, so1,
):
    wid = lax.axis_index("s") * NC + lax.axis_index("c")
    pfs = (pf0, pf1)
    rows = (r0, r1)
    obs = (ob0, ob1)
    sjs = (sj0, sj1)
    sgs = (sg0, sg1)
    sos = (so0, so1)
    xbase = wid * NCH

    def fire(j, p):
        pltpu.make_async_copy(
            xs_hbm.at[xbase + j], jbuf.at[p], sjs[p]
        ).start()
        pltpu.make_async_copy(
            pf_hbm.at[pl.ds((xbase + j) * PFR, PFR), :], pfs[p], sjs[p]
        ).start()

    def start_gather(j, p):
        pltpu.make_async_copy(
            xs_hbm.at[xbase + j], jbuf.at[p], sjs[p]
        ).wait()
        pltpu.make_async_copy(
            pf_hbm.at[pl.ds((xbase + j) * PFR, PFR), :], pfs[p], sjs[p]
        ).wait()
        pltpu.make_async_copy(wp_hbm.at[jbuf.at[p]], rows[p], sgs[p]).start()

    fire(0, 0)
    start_gather(0, 0)
    fire(1, 1)

    def body(i, carry):
        for p in range(NBUF):
            j = NBUF * i + p

            pltpu.make_async_copy(wp_hbm.at[jbuf.at[p]], rows[p], sgs[p]).wait()

            @pl.when(j + 1 < NCH)
            def _():
                start_gather(j + 1, 1 - p)

            @pl.when(j + NBUF < NCH)
            def _():
                fire(j + NBUF, p)

            @pl.when(j >= NBUF)
            def _():
                pltpu.make_async_copy(
                    obs[p],
                    out_hbm.at[pl.ds(0, OPC), :],
                    sos[p],
                ).wait()

            def pack(g, carry2):
                for h in range(8):
                    r = 8 * g + h
                    pv = pfs[p][g, pl.ds(16 * h, 16)]
                    for cc in range(0, DIM, 16):
                        lo = rows[p][r, pl.ds(cc, 16)]
                        hi = rows[p][r, pl.ds(DIM + cc, 16)]
                        obs[p][4 * g + h // 2, pl.ds((h % 2) * DIM + cc, 16)] = (
                            lo + (hi - lo) * pv
                        )
                return carry2

            lax.fori_loop(0, CHUNK // 8, pack, 0)
            pltpu.make_async_copy(
                obs[p],
                out_hbm.at[pl.ds((xbase + j) * OPC, OPC), :],
                sos[p],
            ).start()

        return carry

    lax.fori_loop(0, NCH // NBUF, body, 0)

    for p in range(NBUF):
        j = NCH - NBUF + p
        pltpu.make_async_copy(
            obs[p], out_hbm.at[pl.ds((xbase + j) * OPC, OPC), :], sos[p]
        ).wait()


def kernel(x, weight):
    xi = x.astype(jnp.int32)
    xs2 = (xi >> 1).reshape(BATCH // CHUNK, CHUNK)
    pf = jnp.broadcast_to(
        (xi & 1).astype(jnp.float32).reshape(BATCH, 1), (BATCH, 16)
    ).reshape(BATCH * 16 // CHUNK, CHUNK)
    wp = weight.reshape(NUM_ROWS // 2, 2 * DIM)
    out = _gather(xs2, pf, wp)
    return out.reshape(NA, NT, DIM)
